# baseline (speedup 1.0000x reference)
import functools

import jax
import jax.numpy as jnp
from jax import lax
from jax.experimental import pallas as pl
from jax.experimental.pallas import tpu as pltpu
from jax.experimental.pallas import tpu_sc as plsc

N = 10000
F = 128
HID = 128
H = 8
C = 16
L = 16
NC = 2
NS = 16
NW = NC * NS
K = 128
NB = 320
NBA = 328
NPAD = NW * NB
CH = 90
CAPT = CH * K
RB = 1280
GN = NPAD // RB


def _lane_gather(t, idx):
    return lax.gather(
        t, idx[:, None],
        lax.GatherDimensionNumbers(offset_dims=(), collapsed_slice_dims=(0,),
                                   start_index_map=(0,)),
        (1,), mode=lax.GatherScatterMode.PROMISE_IN_BOUNDS)


def _make_sc_kernel():
    mesh = plsc.VectorSubcoreMesh(
        core_axis_name="c", subcore_axis_name="s", num_cores=NC,
        num_subcores=NS)

    @functools.partial(
        pl.kernel,
        out_type=(
            jax.ShapeDtypeStruct((NPAD, F), jnp.float32),
            jax.ShapeDtypeStruct((NPAD, L), jnp.float32),
        ),
        mesh=mesh,
        scratch_types=[
            pltpu.VMEM((K,), jnp.int32),
            pltpu.VMEM((K,), jnp.int32),
            pltpu.VMEM((K,), jnp.int32),
            pltpu.VMEM((K, F), jnp.float32),
            pltpu.VMEM((K, F), jnp.float32),
            pltpu.VMEM((NBA, F), jnp.float32),
            pltpu.VMEM((NBA, L), jnp.float32),
            pltpu.VMEM((H * C,), jnp.float32),
            pltpu.SemaphoreType.DMA,
            pltpu.SemaphoreType.DMA,
        ],
    )
    def sc_fn(xl_hbm, xr_hbm, att_hbm, src_hbm, dstg_hbm, dstl_hbm,
              acc_out, den_out, src_v, dstg_v, dstl_v, rows_l, rows_r,
              acc_v, den_v, attv, sem1, sem2):
        cid = lax.axis_index("c")
        sid = lax.axis_index("s")
        wid = cid * NS + sid
        z16 = jnp.zeros((L,), jnp.float32)

        pltpu.sync_copy(att_hbm, attv)
        att_vecs = [attv[pl.ds(h * C, C)] for h in range(H)]
        lanes = lax.iota(jnp.int32, L)

        carry0 = (jnp.int32(-1), z16) + tuple(z16 for _ in range(H))

        def _chunk(j, carry):
            pltpu.sync_copy(src_hbm.at[wid, j], src_v)
            pltpu.sync_copy(dstg_hbm.at[wid, j], dstg_v)
            pltpu.sync_copy(dstl_hbm.at[wid, j], dstl_v)
            pltpu.async_copy(xl_hbm.at[src_v], rows_l, sem1).wait()
            pltpu.async_copy(xr_hbm.at[dstg_v], rows_r, sem2).wait()

            def _grp(gg, carry):
                e0 = gg * L
                dl16 = dstl_v[pl.ds(e0, L)]
                prev, dacc = carry[0], carry[1]
                accs = list(carry[2:])
                for i in range(L):
                    e = e0 + i
                    dl = dl16[i]
                    same = dl == prev
                    wv = z16
                    for h in range(H):
                        zl = rows_l[e, pl.ds(h * C, C)]
                        zr = rows_r[e, pl.ds(h * C, C)]
                        z = zl + zr
                        z = jnp.maximum(z, 0.2 * z)
                        t = z * att_vecs[h]
                        for sh in (1, 2, 4, 8):
                            t = t + _lane_gather(t, lanes ^ sh)
                        wh = jnp.exp(t)
                        wv = jnp.where(lanes == h, wh, wv)
                        accs[h] = jnp.where(same, accs[h] + zl * wh, zl * wh)
                        acc_v[dl, pl.ds(h * C, C)] = accs[h]
                    dacc = jnp.where(same, dacc + wv, wv)
                    den_v[dl, :] = dacc
                    prev = dl
                return (prev, dacc) + tuple(accs)

            return lax.fori_loop(0, K // L, _grp, carry)

        lax.fori_loop(0, CH, _chunk, carry0)

        g0 = wid * NB
        pltpu.sync_copy(acc_v.at[pl.ds(0, NB)], acc_out.at[pl.ds(g0, NB)])
        pltpu.sync_copy(den_v.at[pl.ds(0, NB)], den_out.at[pl.ds(g0, NB)])

    return sc_fn


def _pre_call(xp, wl, wr):
    def body(x_ref, wl_ref, wr_ref, xl_ref, xr_ref):
        xb = x_ref[...]
        xl_ref[...] = jnp.dot(xb, wl_ref[...],
                              preferred_element_type=jnp.float32)
        xr_ref[...] = jnp.dot(xb, wr_ref[...],
                              preferred_element_type=jnp.float32)

    return pl.pallas_call(
        body,
        grid=(GN,),
        in_specs=[
            pl.BlockSpec((RB, F), lambda i: (i, 0)),
            pl.BlockSpec((F, HID), lambda i: (0, 0)),
            pl.BlockSpec((F, HID), lambda i: (0, 0)),
        ],
        out_specs=[
            pl.BlockSpec((RB, HID), lambda i: (i, 0)),
            pl.BlockSpec((RB, HID), lambda i: (i, 0)),
        ],
        out_shape=[jax.ShapeDtypeStruct((NPAD, HID), jnp.float32)] * 2,
    )(xp, wl, wr)


def _mid_call(acc, den, b, g, be, resid, wl, wr, last):
    has_res = resid is not None

    def body(*refs):
        it = iter(refs)
        acc_ref = next(it)
        den_ref = next(it)
        b_ref = next(it)
        g_ref = next(it)
        be_ref = next(it)
        res_ref = next(it) if has_res else None
        wl_ref = None if last else next(it)
        wr_ref = None if last else next(it)
        y_ref = next(it)
        xl_ref = None if last else next(it)
        xr_ref = None if last else next(it)

        den16 = den_ref[...]
        ji = lax.broadcasted_iota(jnp.int32, (L, HID), 0)
        ci = lax.broadcasted_iota(jnp.int32, (L, HID), 1)
        em = jnp.where(ci // C == ji, 1.0, 0.0).astype(jnp.float32)
        dfull = jnp.dot(den16, em, preferred_element_type=jnp.float32)
        o = acc_ref[...] / (dfull + 1e-16) + b_ref[...]
        m = jnp.mean(o, axis=-1, keepdims=True)
        v = jnp.mean((o - m) ** 2, axis=-1, keepdims=True)
        yn = (o - m) / jnp.sqrt(v + 1e-5) * g_ref[...] + be_ref[...]
        y = jnp.where(yn > 0, yn, jnp.exp(jnp.minimum(yn, 0.0)) - 1.0)
        if has_res:
            y = y + res_ref[...]
        y_ref[...] = y
        if not last:
            xl_ref[...] = jnp.dot(y, wl_ref[...],
                                  preferred_element_type=jnp.float32)
            xr_ref[...] = jnp.dot(y, wr_ref[...],
                                  preferred_element_type=jnp.float32)

    in_specs = [
        pl.BlockSpec((RB, F), lambda i: (i, 0)),
        pl.BlockSpec((RB, L), lambda i: (i, 0)),
        pl.BlockSpec((1, HID), lambda i: (0, 0)),
        pl.BlockSpec((1, HID), lambda i: (0, 0)),
        pl.BlockSpec((1, HID), lambda i: (0, 0)),
    ]
    args = [acc, den, b, g, be]
    if has_res:
        in_specs.append(pl.BlockSpec((RB, F), lambda i: (i, 0)))
        args.append(resid)
    out_specs = [pl.BlockSpec((RB, HID), lambda i: (i, 0))]
    n_out = N if last else NPAD
    out_shape = [jax.ShapeDtypeStruct((n_out, HID), jnp.float32)]
    if not last:
        in_specs += [pl.BlockSpec((F, HID), lambda i: (0, 0))] * 2
        args += [wl, wr]
        out_specs += [pl.BlockSpec((RB, HID), lambda i: (i, 0))] * 2
        out_shape += [jax.ShapeDtypeStruct((NPAD, HID), jnp.float32)] * 2

    return pl.pallas_call(
        body,
        grid=(GN,),
        in_specs=in_specs,
        out_specs=out_specs,
        out_shape=out_shape,
    )(*args)


def kernel(x, edge_index, Wl1, Wr1, att1, b1, g1, be1, Wl2, Wr2, att2, b2,
           g2, be2, Wl3, Wr3, att3, b3, g3, be3):
    idt = edge_index.dtype
    loop = jnp.arange(N, dtype=idt)
    src = jnp.concatenate([edge_index[0], loop])
    dst = jnp.concatenate([edge_index[1], loop])
    e2 = dst.shape[0]

    perm = jnp.argsort(dst)
    src_s = src[perm]
    dst_s = dst[perm]
    cnt = jnp.bincount(dst, length=N)
    node_start = jnp.cumsum(cnt) - cnt
    bucket = dst_s // NB
    tile_start = node_start[bucket * NB]
    rank = jnp.arange(e2, dtype=idt) - tile_start.astype(idt)
    pos = bucket.astype(idt) * CAPT + rank
    pos = jnp.where(rank < CAPT, pos, NW * CAPT)
    src_p = jnp.zeros((NW * CAPT,), idt).at[pos].set(src_s, mode="drop")
    dstg_p = jnp.zeros((NW * CAPT,), idt).at[pos].set(dst_s, mode="drop")
    dstl_p = jnp.full((NW * CAPT,), NB, idt).at[pos].set(
        (dst_s - bucket * NB).astype(idt), mode="drop")
    src3 = src_p.reshape(NW, CH, K)
    dstg3 = dstg_p.reshape(NW, CH, K)
    dstl3 = dstl_p.reshape(NW, CH, K)

    xp = jnp.pad(x, ((0, NPAD - N), (0, 0)))
    b1r, g1r, be1r = b1.reshape(1, HID), g1.reshape(1, HID), be1.reshape(1, HID)
    b2r, g2r, be2r = b2.reshape(1, HID), g2.reshape(1, HID), be2.reshape(1, HID)
    b3r, g3r, be3r = b3.reshape(1, HID), g3.reshape(1, HID), be3.reshape(1, HID)
    att1f, att2f, att3f = (att1.reshape(-1), att2.reshape(-1),
                           att3.reshape(-1))

    sc = _make_sc_kernel()

    xl, xr = _pre_call(xp, Wl1, Wr1)
    acc, den = sc(xl, xr, att1f, src3, dstg3, dstl3)
    y1, xl2, xr2 = _mid_call(acc, den, b1r, g1r, be1r, None, Wl2, Wr2, False)
    acc2, den2 = sc(xl2, xr2, att2f, src3, dstg3, dstl3)
    y2, xl3, xr3 = _mid_call(acc2, den2, b2r, g2r, be2r, y1, Wl3, Wr3, False)
    acc3, den3 = sc(xl3, xr3, att3f, src3, dstg3, dstl3)
    (y3,) = _mid_call(acc3, den3, b3r, g3r, be3r, y2, None, None, True)
    return y3


# speedup vs baseline: 1.2960x; 1.2960x over previous
import functools

import jax
import jax.numpy as jnp
from jax import lax
from jax.experimental import pallas as pl
from jax.experimental.pallas import tpu as pltpu
from jax.experimental.pallas import tpu_sc as plsc

N = 10000
F = 128
HID = 128
H = 8
C = 16
L = 16
NC = 2
NS = 16
NW = NC * NS
K = 96
NB = 320
NBA = 328
NPAD = NW * NB
CH = 120
CAPT = CH * K
RB = 1280
GN = NPAD // RB


def _lane_gather(t, idx):
    return lax.gather(
        t, idx[:, None],
        lax.GatherDimensionNumbers(offset_dims=(), collapsed_slice_dims=(0,),
                                   start_index_map=(0,)),
        (1,), mode=lax.GatherScatterMode.PROMISE_IN_BOUNDS)


def _make_sc_kernel():
    mesh = plsc.VectorSubcoreMesh(
        core_axis_name="c", subcore_axis_name="s", num_cores=NC,
        num_subcores=NS)

    @functools.partial(
        pl.kernel,
        out_type=(
            jax.ShapeDtypeStruct((NPAD, F), jnp.float32),
            jax.ShapeDtypeStruct((NPAD, L), jnp.float32),
        ),
        mesh=mesh,
        scratch_types=[
            pltpu.VMEM((K,), jnp.int32),
            pltpu.VMEM((K,), jnp.int32),
            pltpu.VMEM((K,), jnp.int32),
            pltpu.VMEM((K, F), jnp.float32),
            pltpu.VMEM((K, F), jnp.float32),
            pltpu.VMEM((K, L), jnp.float32),
            pltpu.VMEM((NBA, F), jnp.float32),
            pltpu.VMEM((NBA, L), jnp.float32),
            pltpu.VMEM((H * C,), jnp.float32),
            pltpu.SemaphoreType.DMA,
            pltpu.SemaphoreType.DMA,
        ],
    )
    def sc_fn(xl_hbm, xr_hbm, att_hbm, src_hbm, dstg_hbm, dstl_hbm,
              acc_out, den_out, src_v, dstg_v, dstl_v, rows_l, rows_r,
              wbuf, acc_v, den_v, attv, sem1, sem2):
        cid = lax.axis_index("c")
        sid = lax.axis_index("s")
        wid = cid * NS + sid
        z16 = jnp.zeros((L,), jnp.float32)

        pltpu.sync_copy(att_hbm, attv)
        att_vecs = [attv[pl.ds(h * C, C)] for h in range(H)]
        lanes = lax.iota(jnp.int32, L)

        carry0 = (jnp.int32(-1), z16) + tuple(z16 for _ in range(H))

        def _chunk(j, carry):
            pltpu.sync_copy(src_hbm.at[wid, j], src_v)
            pltpu.sync_copy(dstg_hbm.at[wid, j], dstg_v)
            pltpu.sync_copy(dstl_hbm.at[wid, j], dstl_v)
            pltpu.async_copy(xl_hbm.at[src_v], rows_l, sem1).wait()
            pltpu.async_copy(xr_hbm.at[dstg_v], rows_r, sem2).wait()

            def _p1(gg, _):
                e0 = gg * L
                for i in range(L):
                    e = e0 + i
                    wv = z16
                    for h in range(H):
                        zl = rows_l[e, pl.ds(h * C, C)]
                        zr = rows_r[e, pl.ds(h * C, C)]
                        z = zl + zr
                        z = jnp.maximum(z, 0.2 * z)
                        t = z * att_vecs[h]
                        for sh in (1, 2, 4, 8):
                            t = t + _lane_gather(t, lanes ^ sh)
                        wh = jnp.exp(t)
                        wv = jnp.where(lanes == h, wh, wv)
                        rows_l[e, pl.ds(h * C, C)] = zl * wh
                    wbuf[e, :] = wv
                return 0

            lax.fori_loop(0, K // L, _p1, 0)

            def _p2(gg, carry):
                e0 = gg * L
                dl16 = dstl_v[pl.ds(e0, L)]
                prev, dacc = carry[0], carry[1]
                accs = list(carry[2:])
                for i in range(L):
                    e = e0 + i
                    dl = dl16[i]
                    same = dl == prev
                    for h in range(H):
                        ch = rows_l[e, pl.ds(h * C, C)]
                        accs[h] = jnp.where(same, accs[h] + ch, ch)
                        acc_v[dl, pl.ds(h * C, C)] = accs[h]
                    wv = wbuf[e, :]
                    dacc = jnp.where(same, dacc + wv, wv)
                    den_v[dl, :] = dacc
                    prev = dl
                return (prev, dacc) + tuple(accs)

            return lax.fori_loop(0, K // L, _p2, carry)

        lax.fori_loop(0, CH, _chunk, carry0)

        g0 = wid * NB
        pltpu.sync_copy(acc_v.at[pl.ds(0, NB)], acc_out.at[pl.ds(g0, NB)])
        pltpu.sync_copy(den_v.at[pl.ds(0, NB)], den_out.at[pl.ds(g0, NB)])

    return sc_fn


def _pre_call(xp, wl, wr):
    def body(x_ref, wl_ref, wr_ref, xl_ref, xr_ref):
        xb = x_ref[...]
        xl_ref[...] = jnp.dot(xb, wl_ref[...],
                              preferred_element_type=jnp.float32)
        xr_ref[...] = jnp.dot(xb, wr_ref[...],
                              preferred_element_type=jnp.float32)

    return pl.pallas_call(
        body,
        grid=(GN,),
        in_specs=[
            pl.BlockSpec((RB, F), lambda i: (i, 0)),
            pl.BlockSpec((F, HID), lambda i: (0, 0)),
            pl.BlockSpec((F, HID), lambda i: (0, 0)),
        ],
        out_specs=[
            pl.BlockSpec((RB, HID), lambda i: (i, 0)),
            pl.BlockSpec((RB, HID), lambda i: (i, 0)),
        ],
        out_shape=[jax.ShapeDtypeStruct((NPAD, HID), jnp.float32)] * 2,
    )(xp, wl, wr)


def _mid_call(acc, den, b, g, be, resid, wl, wr, last):
    has_res = resid is not None

    def body(*refs):
        it = iter(refs)
        acc_ref = next(it)
        den_ref = next(it)
        b_ref = next(it)
        g_ref = next(it)
        be_ref = next(it)
        res_ref = next(it) if has_res else None
        wl_ref = None if last else next(it)
        wr_ref = None if last else next(it)
        y_ref = next(it)
        xl_ref = None if last else next(it)
        xr_ref = None if last else next(it)

        den16 = den_ref[...]
        ji = lax.broadcasted_iota(jnp.int32, (L, HID), 0)
        ci = lax.broadcasted_iota(jnp.int32, (L, HID), 1)
        em = jnp.where(ci // C == ji, 1.0, 0.0).astype(jnp.float32)
        dfull = jnp.dot(den16, em, preferred_element_type=jnp.float32)
        o = acc_ref[...] / (dfull + 1e-16) + b_ref[...]
        m = jnp.mean(o, axis=-1, keepdims=True)
        v = jnp.mean((o - m) ** 2, axis=-1, keepdims=True)
        yn = (o - m) / jnp.sqrt(v + 1e-5) * g_ref[...] + be_ref[...]
        y = jnp.where(yn > 0, yn, jnp.exp(jnp.minimum(yn, 0.0)) - 1.0)
        if has_res:
            y = y + res_ref[...]
        y_ref[...] = y
        if not last:
            xl_ref[...] = jnp.dot(y, wl_ref[...],
                                  preferred_element_type=jnp.float32)
            xr_ref[...] = jnp.dot(y, wr_ref[...],
                                  preferred_element_type=jnp.float32)

    in_specs = [
        pl.BlockSpec((RB, F), lambda i: (i, 0)),
        pl.BlockSpec((RB, L), lambda i: (i, 0)),
        pl.BlockSpec((1, HID), lambda i: (0, 0)),
        pl.BlockSpec((1, HID), lambda i: (0, 0)),
        pl.BlockSpec((1, HID), lambda i: (0, 0)),
    ]
    args = [acc, den, b, g, be]
    if has_res:
        in_specs.append(pl.BlockSpec((RB, F), lambda i: (i, 0)))
        args.append(resid)
    out_specs = [pl.BlockSpec((RB, HID), lambda i: (i, 0))]
    n_out = N if last else NPAD
    out_shape = [jax.ShapeDtypeStruct((n_out, HID), jnp.float32)]
    if not last:
        in_specs += [pl.BlockSpec((F, HID), lambda i: (0, 0))] * 2
        args += [wl, wr]
        out_specs += [pl.BlockSpec((RB, HID), lambda i: (i, 0))] * 2
        out_shape += [jax.ShapeDtypeStruct((NPAD, HID), jnp.float32)] * 2

    return pl.pallas_call(
        body,
        grid=(GN,),
        in_specs=in_specs,
        out_specs=out_specs,
        out_shape=out_shape,
    )(*args)


def kernel(x, edge_index, Wl1, Wr1, att1, b1, g1, be1, Wl2, Wr2, att2, b2,
           g2, be2, Wl3, Wr3, att3, b3, g3, be3):
    idt = edge_index.dtype
    loop = jnp.arange(N, dtype=idt)
    src = jnp.concatenate([edge_index[0], loop])
    dst = jnp.concatenate([edge_index[1], loop])
    e2 = dst.shape[0]

    perm = jnp.argsort(dst)
    src_s = src[perm]
    dst_s = dst[perm]
    cnt = jnp.bincount(dst, length=N)
    node_start = jnp.cumsum(cnt) - cnt
    bucket = dst_s // NB
    tile_start = node_start[bucket * NB]
    rank = jnp.arange(e2, dtype=idt) - tile_start.astype(idt)
    pos = bucket.astype(idt) * CAPT + rank
    pos = jnp.where(rank < CAPT, pos, NW * CAPT)
    src_p = jnp.zeros((NW * CAPT,), idt).at[pos].set(src_s, mode="drop")
    dstg_p = jnp.zeros((NW * CAPT,), idt).at[pos].set(dst_s, mode="drop")
    dstl_p = jnp.full((NW * CAPT,), NB, idt).at[pos].set(
        (dst_s - bucket * NB).astype(idt), mode="drop")
    src3 = src_p.reshape(NW, CH, K)
    dstg3 = dstg_p.reshape(NW, CH, K)
    dstl3 = dstl_p.reshape(NW, CH, K)

    xp = jnp.pad(x, ((0, NPAD - N), (0, 0)))
    b1r, g1r, be1r = b1.reshape(1, HID), g1.reshape(1, HID), be1.reshape(1, HID)
    b2r, g2r, be2r = b2.reshape(1, HID), g2.reshape(1, HID), be2.reshape(1, HID)
    b3r, g3r, be3r = b3.reshape(1, HID), g3.reshape(1, HID), be3.reshape(1, HID)
    att1f, att2f, att3f = (att1.reshape(-1), att2.reshape(-1),
                           att3.reshape(-1))

    sc = _make_sc_kernel()

    xl, xr = _pre_call(xp, Wl1, Wr1)
    acc, den = sc(xl, xr, att1f, src3, dstg3, dstl3)
    y1, xl2, xr2 = _mid_call(acc, den, b1r, g1r, be1r, None, Wl2, Wr2, False)
    acc2, den2 = sc(xl2, xr2, att2f, src3, dstg3, dstl3)
    y2, xl3, xr3 = _mid_call(acc2, den2, b2r, g2r, be2r, y1, Wl3, Wr3, False)
    acc3, den3 = sc(xl3, xr3, att3f, src3, dstg3, dstl3)
    (y3,) = _mid_call(acc3, den3, b3r, g3r, be3r, y2, None, None, True)
    return y3


# speedup vs baseline: 1.5866x; 1.2242x over previous
import functools

import jax
import jax.numpy as jnp
from jax import lax
from jax.experimental import pallas as pl
from jax.experimental.pallas import tpu as pltpu
from jax.experimental.pallas import tpu_sc as plsc

N = 10000
F = 128
HID = 128
H = 8
C = 16
L = 16
NC = 2
NS = 16
NW = NC * NS
K = 48
NB = 320
NBA = 328
NPAD = NW * NB
CH = 240
CAPT = CH * K
RB = 1280
GN = NPAD // RB


def _lane_gather(t, idx):
    return lax.gather(
        t, idx[:, None],
        lax.GatherDimensionNumbers(offset_dims=(), collapsed_slice_dims=(0,),
                                   start_index_map=(0,)),
        (1,), mode=lax.GatherScatterMode.PROMISE_IN_BOUNDS)


def _make_sc_kernel():
    mesh = plsc.VectorSubcoreMesh(
        core_axis_name="c", subcore_axis_name="s", num_cores=NC,
        num_subcores=NS)

    @functools.partial(
        pl.kernel,
        out_type=(
            jax.ShapeDtypeStruct((NPAD, F), jnp.float32),
            jax.ShapeDtypeStruct((NPAD, L), jnp.float32),
        ),
        mesh=mesh,
        scratch_types=[
            pltpu.VMEM((3, K), jnp.int32),
            pltpu.VMEM((3, K), jnp.int32),
            pltpu.VMEM((K,), jnp.int32),
            pltpu.VMEM((K, F), jnp.float32),
            pltpu.VMEM((K, F), jnp.float32),
            pltpu.VMEM((K, F), jnp.float32),
            pltpu.VMEM((K, F), jnp.float32),
            pltpu.VMEM((K, L), jnp.float32),
            pltpu.VMEM((NBA, F), jnp.float32),
            pltpu.VMEM((NBA, L), jnp.float32),
            pltpu.VMEM((H * C,), jnp.float32),
            pltpu.SemaphoreType.DMA,
            pltpu.SemaphoreType.DMA,
            pltpu.SemaphoreType.DMA,
            pltpu.SemaphoreType.DMA,
            pltpu.SemaphoreType.DMA,
            pltpu.SemaphoreType.DMA,
        ],
    )
    def sc_fn(xl_hbm, xr_hbm, att_hbm, idx_hbm, acc_out, den_out,
              idx_a, idx_b, dlbuf, rl_a, rr_a, rl_b, rr_b, wbuf,
              acc_v, den_v, attv, semi_a, semi_b, seml_a, semr_a,
              seml_b, semr_b):
        cid = lax.axis_index("c")
        sid = lax.axis_index("s")
        wid = cid * NS + sid
        z16 = jnp.zeros((L,), jnp.float32)

        pltpu.sync_copy(att_hbm, attv)
        att_vecs = [attv[pl.ds(h * C, C)] for h in range(H)]
        lanes = lax.iota(jnp.int32, L)

        carry0 = (jnp.int32(-1), z16) + tuple(z16 for _ in range(H))

        def _compute(rl, rr, carry):
            def _p1(gg, _):
                e0 = gg * L
                for i in range(L):
                    e = e0 + i
                    wv = z16
                    for h in range(H):
                        zl = rl[e, pl.ds(h * C, C)]
                        zr = rr[e, pl.ds(h * C, C)]
                        z = zl + zr
                        z = jnp.maximum(z, 0.2 * z)
                        t = z * att_vecs[h]
                        for sh in (1, 2, 4, 8):
                            t = t + _lane_gather(t, lanes ^ sh)
                        wh = jnp.exp(t)
                        wv = jnp.where(lanes == h, wh, wv)
                        rl[e, pl.ds(h * C, C)] = zl * wh
                    wbuf[e, :] = wv
                return 0

            lax.fori_loop(0, K // L, _p1, 0)

            def _p2(gg, carry):
                e0 = gg * L
                dl16 = dlbuf[pl.ds(e0, L)]
                prev, dacc = carry[0], carry[1]
                accs = list(carry[2:])
                for i in range(L):
                    e = e0 + i
                    dl = dl16[i]
                    same = dl == prev
                    for h in range(H):
                        ch = rl[e, pl.ds(h * C, C)]
                        accs[h] = jnp.where(same, accs[h] + ch, ch)
                        acc_v[dl, pl.ds(h * C, C)] = accs[h]
                    wv = wbuf[e, :]
                    dacc = jnp.where(same, dacc + wv, wv)
                    den_v[dl, :] = dacc
                    prev = dl
                return (prev, dacc) + tuple(accs)

            return lax.fori_loop(0, K // L, _p2, carry)

        def _body(kk, carry, idx_p, idx_q, rl_p, rr_p, rl_q, rr_q,
                  semi_p, semi_q, seml_p, semr_p, seml_q, semr_q, par):
            j = 2 * kk + par
            pltpu.make_async_copy(xl_hbm.at[idx_p.at[0]], rl_p,
                                  seml_p).wait()
            pltpu.make_async_copy(xr_hbm.at[idx_p.at[1]], rr_p,
                                  semr_p).wait()
            for g in range(K // L):
                dlbuf[pl.ds(g * L, L)] = idx_p[2, pl.ds(g * L, L)]
            pltpu.make_async_copy(idx_hbm.at[wid, j], idx_q, semi_q).wait()
            pltpu.async_copy(xl_hbm.at[idx_q.at[0]], rl_q, seml_q)
            pltpu.async_copy(xr_hbm.at[idx_q.at[1]], rr_q, semr_q)
            jn2 = jnp.minimum(j + 2, CH - 1)
            pltpu.async_copy(idx_hbm.at[wid, jn2], idx_p, semi_p)
            return _compute(rl_p, rr_p, carry)

        def _pair(kk, carry):
            carry = _body(kk, carry, idx_a, idx_b, rl_a, rr_a, rl_b, rr_b,
                          semi_a, semi_b, seml_a, semr_a, seml_b, semr_b, 0)
            carry = _body(kk, carry, idx_b, idx_a, rl_b, rr_b, rl_a, rr_a,
                          semi_b, semi_a, seml_b, semr_b, seml_a, semr_a, 1)
            return carry

        pltpu.sync_copy(idx_hbm.at[wid, 0], idx_a)
        pltpu.async_copy(idx_hbm.at[wid, 1], idx_b, semi_b)
        pltpu.async_copy(xl_hbm.at[idx_a.at[0]], rl_a, seml_a)
        pltpu.async_copy(xr_hbm.at[idx_a.at[1]], rr_a, semr_a)

        lax.fori_loop(0, CH // 2, _pair, carry0)

        pltpu.make_async_copy(xl_hbm.at[idx_a.at[0]], rl_a, seml_a).wait()
        pltpu.make_async_copy(xr_hbm.at[idx_a.at[1]], rr_a, semr_a).wait()
        pltpu.make_async_copy(idx_hbm.at[wid, 0], idx_b, semi_b).wait()

        g0 = wid * NB
        pltpu.sync_copy(acc_v.at[pl.ds(0, NB)], acc_out.at[pl.ds(g0, NB)])
        pltpu.sync_copy(den_v.at[pl.ds(0, NB)], den_out.at[pl.ds(g0, NB)])

    return sc_fn


def _pre_call(xp, wl, wr):
    def body(x_ref, wl_ref, wr_ref, xl_ref, xr_ref):
        xb = x_ref[...]
        xl_ref[...] = jnp.dot(xb, wl_ref[...],
                              preferred_element_type=jnp.float32)
        xr_ref[...] = jnp.dot(xb, wr_ref[...],
                              preferred_element_type=jnp.float32)

    return pl.pallas_call(
        body,
        grid=(GN,),
        in_specs=[
            pl.BlockSpec((RB, F), lambda i: (i, 0)),
            pl.BlockSpec((F, HID), lambda i: (0, 0)),
            pl.BlockSpec((F, HID), lambda i: (0, 0)),
        ],
        out_specs=[
            pl.BlockSpec((RB, HID), lambda i: (i, 0)),
            pl.BlockSpec((RB, HID), lambda i: (i, 0)),
        ],
        out_shape=[jax.ShapeDtypeStruct((NPAD, HID), jnp.float32)] * 2,
    )(xp, wl, wr)


def _mid_call(acc, den, b, g, be, resid, wl, wr, last):
    has_res = resid is not None

    def body(*refs):
        it = iter(refs)
        acc_ref = next(it)
        den_ref = next(it)
        b_ref = next(it)
        g_ref = next(it)
        be_ref = next(it)
        res_ref = next(it) if has_res else None
        wl_ref = None if last else next(it)
        wr_ref = None if last else next(it)
        y_ref = next(it)
        xl_ref = None if last else next(it)
        xr_ref = None if last else next(it)

        den16 = den_ref[...]
        ji = lax.broadcasted_iota(jnp.int32, (L, HID), 0)
        ci = lax.broadcasted_iota(jnp.int32, (L, HID), 1)
        em = jnp.where(ci // C == ji, 1.0, 0.0).astype(jnp.float32)
        dfull = jnp.dot(den16, em, preferred_element_type=jnp.float32)
        o = acc_ref[...] / (dfull + 1e-16) + b_ref[...]
        m = jnp.mean(o, axis=-1, keepdims=True)
        v = jnp.mean((o - m) ** 2, axis=-1, keepdims=True)
        yn = (o - m) / jnp.sqrt(v + 1e-5) * g_ref[...] + be_ref[...]
        y = jnp.where(yn > 0, yn, jnp.exp(jnp.minimum(yn, 0.0)) - 1.0)
        if has_res:
            y = y + res_ref[...]
        y_ref[...] = y
        if not last:
            xl_ref[...] = jnp.dot(y, wl_ref[...],
                                  preferred_element_type=jnp.float32)
            xr_ref[...] = jnp.dot(y, wr_ref[...],
                                  preferred_element_type=jnp.float32)

    in_specs = [
        pl.BlockSpec((RB, F), lambda i: (i, 0)),
        pl.BlockSpec((RB, L), lambda i: (i, 0)),
        pl.BlockSpec((1, HID), lambda i: (0, 0)),
        pl.BlockSpec((1, HID), lambda i: (0, 0)),
        pl.BlockSpec((1, HID), lambda i: (0, 0)),
    ]
    args = [acc, den, b, g, be]
    if has_res:
        in_specs.append(pl.BlockSpec((RB, F), lambda i: (i, 0)))
        args.append(resid)
    out_specs = [pl.BlockSpec((RB, HID), lambda i: (i, 0))]
    n_out = N if last else NPAD
    out_shape = [jax.ShapeDtypeStruct((n_out, HID), jnp.float32)]
    if not last:
        in_specs += [pl.BlockSpec((F, HID), lambda i: (0, 0))] * 2
        args += [wl, wr]
        out_specs += [pl.BlockSpec((RB, HID), lambda i: (i, 0))] * 2
        out_shape += [jax.ShapeDtypeStruct((NPAD, HID), jnp.float32)] * 2

    return pl.pallas_call(
        body,
        grid=(GN,),
        in_specs=in_specs,
        out_specs=out_specs,
        out_shape=out_shape,
    )(*args)


def kernel(x, edge_index, Wl1, Wr1, att1, b1, g1, be1, Wl2, Wr2, att2, b2,
           g2, be2, Wl3, Wr3, att3, b3, g3, be3):
    idt = edge_index.dtype
    loop = jnp.arange(N, dtype=idt)
    src = jnp.concatenate([edge_index[0], loop])
    dst = jnp.concatenate([edge_index[1], loop])
    e2 = dst.shape[0]

    perm = jnp.argsort(dst)
    src_s = src[perm]
    dst_s = dst[perm]
    cnt = jnp.bincount(dst, length=N)
    node_start = jnp.cumsum(cnt) - cnt
    bucket = dst_s // NB
    tile_start = node_start[bucket * NB]
    rank = jnp.arange(e2, dtype=idt) - tile_start.astype(idt)
    pos = bucket.astype(idt) * CAPT + rank
    pos = jnp.where(rank < CAPT, pos, NW * CAPT)
    src_p = jnp.zeros((NW * CAPT,), idt).at[pos].set(src_s, mode="drop")
    dstg_p = jnp.zeros((NW * CAPT,), idt).at[pos].set(dst_s, mode="drop")
    dstl_p = jnp.full((NW * CAPT,), NB, idt).at[pos].set(
        (dst_s - bucket * NB).astype(idt), mode="drop")
    idx3 = jnp.stack([src_p.reshape(NW, CH, K),
                      dstg_p.reshape(NW, CH, K),
                      dstl_p.reshape(NW, CH, K)], axis=2)

    xp = jnp.pad(x, ((0, NPAD - N), (0, 0)))
    b1r, g1r, be1r = b1.reshape(1, HID), g1.reshape(1, HID), be1.reshape(1, HID)
    b2r, g2r, be2r = b2.reshape(1, HID), g2.reshape(1, HID), be2.reshape(1, HID)
    b3r, g3r, be3r = b3.reshape(1, HID), g3.reshape(1, HID), be3.reshape(1, HID)
    att1f, att2f, att3f = (att1.reshape(-1), att2.reshape(-1),
                           att3.reshape(-1))

    sc = _make_sc_kernel()

    xl, xr = _pre_call(xp, Wl1, Wr1)
    acc, den = sc(xl, xr, att1f, idx3)
    y1, xl2, xr2 = _mid_call(acc, den, b1r, g1r, be1r, None, Wl2, Wr2, False)
    acc2, den2 = sc(xl2, xr2, att2f, idx3)
    y2, xl3, xr3 = _mid_call(acc2, den2, b2r, g2r, be2r, y1, Wl3, Wr3, False)
    acc3, den3 = sc(xl3, xr3, att3f, idx3)
    (y3,) = _mid_call(acc3, den3, b3r, g3r, be3r, y2, None, None, True)
    return y3
